# baseline (speedup 1.0000x reference)
import functools

import jax
import jax.numpy as jnp
from jax import lax
from jax.experimental import pallas as pl
from jax.experimental.pallas import tpu as pltpu
from jax.experimental.pallas import tpu_sc as plsc

N_NODES = 100000
D = 128
E = 250000

NUM_WORKERS = 32
W = 64
B_FAST = 302
B_SLOW = 190
FAST_CORE = 0
CH_F = W * B_FAST
CH_S = W * B_SLOW
B_PAD = 16 * CH_F + 16 * CH_S + (CH_F - CH_S)


def _sc_score(src_idx, dst_idx, emb_user, emb_item, rel):
    mesh = plsc.VectorSubcoreMesh(core_axis_name="c", subcore_axis_name="s")

    @functools.partial(
        pl.kernel,
        mesh=mesh,
        out_type=jax.ShapeDtypeStruct((B_PAD,), jnp.float32),
        scratch_types=[
            pltpu.VMEM((CH_F,), jnp.int32),
            pltpu.VMEM((CH_F,), jnp.int32),
            pltpu.VMEM((W, D // 2), jnp.int32),
            pltpu.VMEM((W, D // 2), jnp.int32),
            pltpu.VMEM((W, D // 2), jnp.int32),
            pltpu.VMEM((W, D // 2), jnp.int32),
            pltpu.VMEM((CH_F,), jnp.float32),
            pltpu.VMEM((D,), jnp.float32),
            pltpu.VMEM((16, 16), jnp.float32),
            pltpu.SemaphoreType.DMA,
            pltpu.SemaphoreType.DMA,
        ],
        compiler_params=pltpu.CompilerParams(
            needs_layout_passes=False, use_tc_tiling_on_sc=False),
    )
    def scorer(src_h, dst_h, user_h, item_h, rel_h, out_h,
               src_v, dst_v, u0, i0, u1, i1, out_v, rel_v, tbuf,
               sem0, sem1):
        cidx = lax.axis_index("c")
        sidx = lax.axis_index("s")
        is_fast = cidx == FAST_CORE
        nblk = jnp.where(is_fast, B_FAST, B_SLOW)
        base = jnp.where(is_fast, sidx * CH_F, 16 * CH_F + sidx * CH_S)
        pltpu.sync_copy(src_h.at[pl.ds(base, CH_F)], src_v)
        pltpu.sync_copy(dst_h.at[pl.ds(base, CH_F)], dst_v)
        pltpu.sync_copy(rel_h, rel_v)
        kv = [rel_v[pl.ds(j * 16, 16)] for j in range(D // 16)]
        row_iota = lax.iota(jnp.int32, 16)

        def copies(b, u, i, sem):
            eb = b * W
            return (
                pltpu.make_async_copy(
                    user_h.at[src_v.at[pl.ds(eb, W)]], u, sem),
                pltpu.make_async_copy(
                    item_h.at[dst_v.at[pl.ds(eb, W)]], i, sem),
            )

        def start_pair(b, u, i, sem):
            for c in copies(b, u, i, sem):
                c.start()

        def wait_pair(b, u, i, sem):
            for c in copies(b, u, i, sem):
                c.wait()

        def compute(b, u, i):
            eb = b * W

            def group(g, carry):
                ebase = g * 16

                def chunk(j, accs):
                    ka = rel_v[pl.ds(j * 32, 16)]
                    kb = rel_v[pl.ds(j * 32 + 16, 16)]
                    new = []
                    for e_off in range(16):
                        uw = u[ebase + e_off, pl.ds(j * 16, 16)]
                        iw = i[ebase + e_off, pl.ds(j * 16, 16)]
                        ua, ub = plsc.unpack(
                            plsc.bitcast(uw, jnp.bfloat16),
                            format=plsc.PackFormat.INTERLEAVED)
                        ia, ib = plsc.unpack(
                            plsc.bitcast(iw, jnp.bfloat16),
                            format=plsc.PackFormat.INTERLEAVED)
                        new.append(accs[e_off]
                                   + (ua * ia) * ka + (ub * ib) * kb)
                    return new

                zero = jnp.zeros((16,), jnp.float32)
                accs = lax.fori_loop(0, D // 32, chunk, [zero] * 16)
                for e_off in range(16):
                    tbuf[e_off, :] = accs[e_off]
                col_sum = plsc.load_gather(
                    tbuf, [row_iota, jnp.zeros((16,), jnp.int32)])
                for c in range(1, 16):
                    col_sum = col_sum + plsc.load_gather(
                        tbuf, [row_iota, jnp.full((16,), c, jnp.int32)])
                out_v[pl.ds(eb + ebase, 16)] = col_sum
                return carry

            lax.fori_loop(0, W // 16, group, 0)

        start_pair(0, u0, i0, sem0)

        def block_pair(ib, carry):
            b0 = 2 * ib
            b1 = b0 + 1
            wait_pair(b0, u0, i0, sem0)
            start_pair(b1, u1, i1, sem1)
            compute(b0, u0, i0)
            wait_pair(b1, u1, i1, sem1)
            start_pair(jnp.minimum(b1 + 1, nblk - 1), u0, i0, sem0)
            compute(b1, u1, i1)
            return carry

        lax.fori_loop(0, nblk // 2, block_pair, 0)
        wait_pair(nblk - 1, u0, i0, sem0)
        pltpu.sync_copy(out_v.at[pl.ds(0, CH_S)],
                        out_h.at[pl.ds(base, CH_S)])

        @pl.when(is_fast)
        def _():
            pltpu.sync_copy(out_v.at[pl.ds(CH_S, CH_F - CH_S)],
                            out_h.at[pl.ds(base + CH_S, CH_F - CH_S)])

    return scorer(src_idx, dst_idx, emb_user, emb_item, rel)


def kernel(edge_pos, edge_neg, emb_user, emb_item, relation_embedding):
    src = jnp.concatenate([edge_pos[0], edge_neg[0]])
    dst = jnp.concatenate([edge_pos[1], edge_neg[1]])
    pad = B_PAD - 2 * E
    src = jnp.pad(src, (0, pad))
    dst = jnp.pad(dst, (0, pad))
    u32 = lax.bitcast_convert_type(
        emb_user.astype(jnp.bfloat16).reshape(N_NODES, D // 2, 2), jnp.int32)
    i32 = lax.bitcast_convert_type(
        emb_item.astype(jnp.bfloat16).reshape(N_NODES, D // 2, 2), jnp.int32)
    rel = (relation_embedding.reshape(D // 32, 16, 2)
           .transpose(0, 2, 1).reshape(D))
    scores = _sc_score(src, dst, u32, i32, rel)
    return scores[:E], scores[E:2 * E]


# speedup vs baseline: 2.5062x; 2.5062x over previous
import functools

import jax
import jax.numpy as jnp
from jax import lax
from jax.experimental import pallas as pl
from jax.experimental.pallas import tpu as pltpu
from jax.experimental.pallas import tpu_sc as plsc

N_NODES = 100000
D = 128
E = 250000

NUM_WORKERS = 32
W = 64
B_FAST = 302
B_SLOW = 190
FAST_CORE = 0
CH_F = W * B_FAST
CH_S = W * B_SLOW
B_PAD = 16 * CH_F + 16 * CH_S + (CH_F - CH_S)


def _sc_score(src_idx, dst_idx, emb_user, emb_item, rel):
    mesh = plsc.VectorSubcoreMesh(core_axis_name="c", subcore_axis_name="s")

    @functools.partial(
        pl.kernel,
        mesh=mesh,
        out_type=jax.ShapeDtypeStruct((B_PAD,), jnp.float32),
        scratch_types=[
            pltpu.VMEM((CH_F,), jnp.int32),
            pltpu.VMEM((CH_F,), jnp.int32),
            pltpu.VMEM((W, D), jnp.float32),
            pltpu.VMEM((W, D), jnp.float32),
            pltpu.VMEM((W, D), jnp.float32),
            pltpu.VMEM((W, D), jnp.float32),
            pltpu.VMEM((CH_F,), jnp.float32),
            pltpu.VMEM((D,), jnp.float32),
            pltpu.VMEM((16, 16), jnp.float32),
            pltpu.SemaphoreType.DMA,
            pltpu.SemaphoreType.DMA,
        ],
        compiler_params=pltpu.CompilerParams(needs_layout_passes=False),
    )
    def scorer(src_h, dst_h, user_h, item_h, rel_h, out_h,
               src_v, dst_v, u0, i0, u1, i1, out_v, rel_v, tbuf,
               sem0, sem1):
        cidx = lax.axis_index("c")
        sidx = lax.axis_index("s")
        is_fast = cidx == FAST_CORE
        nblk = jnp.where(is_fast, B_FAST, B_SLOW)
        base = jnp.where(is_fast, sidx * CH_F, 16 * CH_F + sidx * CH_S)
        pltpu.sync_copy(src_h.at[pl.ds(base, CH_F)], src_v)
        pltpu.sync_copy(dst_h.at[pl.ds(base, CH_F)], dst_v)
        pltpu.sync_copy(rel_h, rel_v)
        kv = [rel_v[pl.ds(j * 16, 16)] for j in range(D // 16)]
        row_iota = lax.iota(jnp.int32, 16)

        def copies(b, u, i, sem):
            eb = b * W
            return (
                pltpu.make_async_copy(
                    user_h.at[src_v.at[pl.ds(eb, W)]], u, sem),
                pltpu.make_async_copy(
                    item_h.at[dst_v.at[pl.ds(eb, W)]], i, sem),
            )

        def start_pair(b, u, i, sem):
            for c in copies(b, u, i, sem):
                c.start()

        def wait_pair(b, u, i, sem):
            for c in copies(b, u, i, sem):
                c.wait()

        def compute(b, u, i):
            eb = b * W

            def group(g, carry):
                ebase = g * 16

                def chunk(j, accs):
                    kvj = rel_v[pl.ds(j * 16, 16)]
                    return [
                        accs[e_off]
                        + (u[ebase + e_off, pl.ds(j * 16, 16)]
                           * i[ebase + e_off, pl.ds(j * 16, 16)]) * kvj
                        for e_off in range(16)
                    ]

                zero = jnp.zeros((16,), jnp.float32)
                accs = lax.fori_loop(0, D // 16, chunk, [zero] * 16)
                for e_off in range(16):
                    tbuf[e_off, :] = accs[e_off]
                col_sum = plsc.load_gather(
                    tbuf, [row_iota, jnp.zeros((16,), jnp.int32)])
                for c in range(1, 16):
                    col_sum = col_sum + plsc.load_gather(
                        tbuf, [row_iota, jnp.full((16,), c, jnp.int32)])
                out_v[pl.ds(eb + ebase, 16)] = col_sum
                return carry

            lax.fori_loop(0, W // 16, group, 0)

        start_pair(0, u0, i0, sem0)

        def block_pair(ib, carry):
            b0 = 2 * ib
            b1 = b0 + 1
            wait_pair(b0, u0, i0, sem0)
            start_pair(b1, u1, i1, sem1)
            compute(b0, u0, i0)
            wait_pair(b1, u1, i1, sem1)
            start_pair(jnp.minimum(b1 + 1, nblk - 1), u0, i0, sem0)
            compute(b1, u1, i1)
            return carry

        lax.fori_loop(0, nblk // 2, block_pair, 0)
        wait_pair(nblk - 1, u0, i0, sem0)
        pltpu.sync_copy(out_v.at[pl.ds(0, CH_S)],
                        out_h.at[pl.ds(base, CH_S)])

        @pl.when(is_fast)
        def _():
            pltpu.sync_copy(out_v.at[pl.ds(CH_S, CH_F - CH_S)],
                            out_h.at[pl.ds(base + CH_S, CH_F - CH_S)])

    return scorer(src_idx, dst_idx, emb_user, emb_item, rel)


def kernel(edge_pos, edge_neg, emb_user, emb_item, relation_embedding):
    src = jnp.concatenate([edge_pos[0], edge_neg[0]])
    dst = jnp.concatenate([edge_pos[1], edge_neg[1]])
    pad = B_PAD - 2 * E
    src = jnp.pad(src, (0, pad))
    dst = jnp.pad(dst, (0, pad))
    rel = relation_embedding.reshape(D)
    scores = _sc_score(src, dst, emb_user, emb_item, rel)
    return scores[:E], scores[E:2 * E]


# speedup vs baseline: 2.5148x; 1.0034x over previous
import functools

import jax
import jax.numpy as jnp
from jax import lax
from jax.experimental import pallas as pl
from jax.experimental.pallas import tpu as pltpu
from jax.experimental.pallas import tpu_sc as plsc

N_NODES = 100000
D = 128
E = 250000

NUM_WORKERS = 32
W = 64
B_FAST = 302
B_SLOW = 190
FAST_CORE = 0
CH_F = W * B_FAST
CH_S = W * B_SLOW
B_PAD = 16 * CH_F + 16 * CH_S + (CH_F - CH_S)


def _sc_score(src_idx, dst_idx, emb_user, emb_item, rel):
    mesh = plsc.VectorSubcoreMesh(core_axis_name="c", subcore_axis_name="s")

    @functools.partial(
        pl.kernel,
        mesh=mesh,
        out_type=jax.ShapeDtypeStruct((B_PAD,), jnp.float32),
        scratch_types=[
            pltpu.VMEM((CH_F,), jnp.int32),
            pltpu.VMEM((CH_F,), jnp.int32),
            pltpu.VMEM((W, D), jnp.float32),
            pltpu.VMEM((W, D), jnp.float32),
            pltpu.VMEM((W, D), jnp.float32),
            pltpu.VMEM((W, D), jnp.float32),
            pltpu.VMEM((CH_F,), jnp.float32),
            pltpu.VMEM((D,), jnp.float32),
            pltpu.VMEM((16, 16), jnp.float32),
            pltpu.SemaphoreType.DMA,
            pltpu.SemaphoreType.DMA,
        ],
        compiler_params=pltpu.CompilerParams(needs_layout_passes=False),
    )
    def scorer(src_h, dst_h, user_h, item_h, rel_h, out_h,
               src_v, dst_v, u0, i0, u1, i1, out_v, rel_v, tbuf,
               sem0, sem1):
        cidx = lax.axis_index("c")
        sidx = lax.axis_index("s")
        is_fast = cidx == FAST_CORE
        nblk = jnp.where(is_fast, B_FAST, B_SLOW)
        base = jnp.where(is_fast, sidx * CH_F, 16 * CH_F + sidx * CH_S)
        pltpu.sync_copy(src_h.at[pl.ds(base, CH_F)], src_v)
        pltpu.sync_copy(dst_h.at[pl.ds(base, CH_F)], dst_v)
        pltpu.sync_copy(rel_h, rel_v)
        row_iota = lax.iota(jnp.int32, 16)

        def copies(b, u, i, sem):
            eb = b * W
            return (
                pltpu.make_async_copy(
                    user_h.at[src_v.at[pl.ds(eb, W)]], u, sem),
                pltpu.make_async_copy(
                    item_h.at[dst_v.at[pl.ds(eb, W)]], i, sem),
            )

        def start_pair(b, u, i, sem):
            for c in copies(b, u, i, sem):
                c.start()

        def wait_pair(b, u, i, sem):
            for c in copies(b, u, i, sem):
                c.wait()

        def compute(b, u, i):
            eb = b * W

            def group(g, carry):
                ebase = g * 16

                def chunk(j, accs):
                    kvj = rel_v[pl.ds(j * 16, 16)]
                    return [
                        accs[e_off]
                        + (u[ebase + e_off, pl.ds(j * 16, 16)]
                           * i[ebase + e_off, pl.ds(j * 16, 16)]) * kvj
                        for e_off in range(16)
                    ]

                zero = jnp.zeros((16,), jnp.float32)
                accs = lax.fori_loop(0, D // 16, chunk, [zero] * 16)
                for e_off in range(16):
                    tbuf[e_off, :] = accs[e_off]
                col_sum = plsc.load_gather(
                    tbuf, [row_iota, jnp.zeros((16,), jnp.int32)])
                for c in range(1, 16):
                    col_sum = col_sum + plsc.load_gather(
                        tbuf, [row_iota, jnp.full((16,), c, jnp.int32)])
                out_v[pl.ds(eb + ebase, 16)] = col_sum
                return carry

            lax.fori_loop(0, W // 16, group, 0)

        start_pair(0, u0, i0, sem0)

        def block_pair(ib, carry):
            b0 = 2 * ib
            b1 = b0 + 1
            wait_pair(b0, u0, i0, sem0)
            start_pair(b1, u1, i1, sem1)
            compute(b0, u0, i0)
            wait_pair(b1, u1, i1, sem1)
            start_pair(jnp.minimum(b1 + 1, nblk - 1), u0, i0, sem0)
            compute(b1, u1, i1)
            return carry

        lax.fori_loop(0, nblk // 2, block_pair, 0)
        wait_pair(nblk - 1, u0, i0, sem0)
        pltpu.sync_copy(out_v.at[pl.ds(0, CH_S)],
                        out_h.at[pl.ds(base, CH_S)])

        @pl.when(is_fast)
        def _():
            pltpu.sync_copy(out_v.at[pl.ds(CH_S, CH_F - CH_S)],
                            out_h.at[pl.ds(base + CH_S, CH_F - CH_S)])

    return scorer(src_idx, dst_idx, emb_user, emb_item, rel)


def kernel(edge_pos, edge_neg, emb_user, emb_item, relation_embedding):
    src = jnp.concatenate([edge_pos[0], edge_neg[0]])
    dst = jnp.concatenate([edge_pos[1], edge_neg[1]])
    pad = B_PAD - 2 * E
    src = jnp.pad(src, (0, pad))
    dst = jnp.pad(dst, (0, pad))
    rel = relation_embedding.reshape(D)
    scores = _sc_score(src, dst, emb_user, emb_item, rel)
    return scores[:E], scores[E:2 * E]
